# baseline (speedup 1.0000x reference)
import functools

import jax
import jax.numpy as jnp
from jax import lax
from jax.experimental import pallas as pl
from jax.experimental.pallas import tpu as pltpu
from jax.experimental.pallas import tpu_sc as plsc

N = 10000
E = 320000
D = 128
H = 128
C = 10

NC = 2
NS = 16
NW = NC * NS
CHUNK = 128
JMAX = 80
EPAD = JMAX * NW * CHUNK
NPAD = 10240
PADROW = 10016
TROWS = NPAD // NS
RCH = TROWS // CHUNK

BNP = 640
GRIDP = NPAD // BNP
BN = 1000
GRID = N // BN

_mesh = plsc.VectorSubcoreMesh(
    core_axis_name="c", subcore_axis_name="s", num_cores=NC, num_subcores=NS
)


@functools.partial(
    pl.kernel,
    out_type=jax.ShapeDtypeStruct((NC, NPAD, H), jnp.float32),
    mesh=_mesh,
    scratch_types=[
        pltpu.VMEM((CHUNK,), jnp.int32),
        pltpu.VMEM((CHUNK,), jnp.int32),
        pltpu.VMEM((CHUNK, H), jnp.float32),
        pltpu.VMEM_SHARED((NPAD, H), jnp.float32),
    ],
)
def _deg_kernel(idx_hbm, ones_hbm, zero_hbm, rowid_hbm, out_hbm,
                eidx0, zidx, ones_v, acc):
    cid = lax.axis_index("c")
    sid = lax.axis_index("s")
    wid = sid * NC + cid
    base = sid * TROWS
    pltpu.sync_copy(zero_hbm, ones_v.at[pl.ds(0, CHUNK)])
    for r in range(RCH):
        pltpu.sync_copy(rowid_hbm.at[pl.ds(base + r * CHUNK, CHUNK)], zidx)
        pltpu.sync_copy(ones_v.at[pl.ds(0, CHUNK)], acc.at[zidx])
    pltpu.sync_copy(ones_hbm, ones_v)
    plsc.subcore_barrier()

    def body(j, carry):
        off = (j * NW + wid) * CHUNK
        pltpu.sync_copy(idx_hbm.at[pl.ds(off, CHUNK)], eidx0)
        pltpu.sync_copy(ones_v, acc.at[eidx0], add=True)
        return carry

    lax.fori_loop(0, JMAX, body, 0)
    plsc.subcore_barrier()
    for r in range(RCH):
        pltpu.sync_copy(rowid_hbm.at[pl.ds(base + r * CHUNK, CHUNK)], zidx)
        pltpu.sync_copy(acc.at[zidx], ones_v.at[pl.ds(0, CHUNK)])
        pltpu.sync_copy(ones_v.at[pl.ds(0, CHUNK)], out_hbm.at[cid, pl.ds(base + r * CHUNK, CHUNK)])


@functools.partial(
    pl.kernel,
    out_type=jax.ShapeDtypeStruct((NC, NPAD, H), jnp.float32),
    mesh=_mesh,
    scratch_types=[
        pltpu.VMEM((CHUNK,), jnp.int32),
        pltpu.VMEM((CHUNK,), jnp.int32),
        pltpu.VMEM((CHUNK,), jnp.int32),
        pltpu.VMEM((CHUNK, H), jnp.float32),
        pltpu.VMEM_SHARED((NPAD, H), jnp.float32),
        pltpu.SemaphoreType.DMA,
    ],
)
def _agg_kernel(g_hbm, src_hbm, dst_hbm, zero_hbm, rowid_hbm, out_hbm,
                sidx, didx, zidx, rows_v, acc, gsem):
    cid = lax.axis_index("c")
    sid = lax.axis_index("s")
    wid = sid * NC + cid
    base = sid * TROWS
    pltpu.sync_copy(zero_hbm, rows_v)
    for r in range(RCH):
        pltpu.sync_copy(rowid_hbm.at[pl.ds(base + r * CHUNK, CHUNK)], zidx)
        pltpu.sync_copy(rows_v, acc.at[zidx])
    plsc.subcore_barrier()

    def body(j, carry):
        off = (j * NW + wid) * CHUNK
        pltpu.sync_copy(src_hbm.at[pl.ds(off, CHUNK)], sidx)
        pltpu.sync_copy(dst_hbm.at[pl.ds(off, CHUNK)], didx)
        pltpu.async_copy(g_hbm.at[sidx], rows_v, gsem).wait()
        pltpu.sync_copy(rows_v, acc.at[didx], add=True)
        return carry

    lax.fori_loop(0, JMAX, body, 0)
    plsc.subcore_barrier()
    for r in range(RCH):
        pltpu.sync_copy(rowid_hbm.at[pl.ds(base + r * CHUNK, CHUNK)], zidx)
        pltpu.sync_copy(acc.at[zidx], rows_v.at[pl.ds(0, CHUNK)])
        pltpu.sync_copy(rows_v.at[pl.ds(0, CHUNK)], out_hbm.at[cid, pl.ds(base + r * CHUNK, CHUNK)])


def _prep_body(dego_ref, degi_ref, x_ref, no_ref, ni_ref, g_ref):
    do = dego_ref[0, :, :1] + dego_ref[1, :, :1]
    di = degi_ref[0, :, :1] + degi_ref[1, :, :1]
    no = jnp.where(do > 0, lax.rsqrt(jnp.maximum(do, 1.0)), 0.0)
    ni = jnp.where(di > 0, lax.rsqrt(jnp.maximum(di, 1.0)), 0.0)
    no_ref[...] = no
    ni_ref[...] = ni
    g_ref[...] = x_ref[...] * no


_prep = pl.pallas_call(
    _prep_body,
    grid=(GRIDP,),
    in_specs=[
        pl.BlockSpec((NC, BNP, H), lambda i: (0, i, 0)),
        pl.BlockSpec((NC, BNP, H), lambda i: (0, i, 0)),
        pl.BlockSpec((BNP, D), lambda i: (i, 0)),
    ],
    out_specs=[
        pl.BlockSpec((BNP, 1), lambda i: (i, 0)),
        pl.BlockSpec((BNP, 1), lambda i: (i, 0)),
        pl.BlockSpec((BNP, D), lambda i: (i, 0)),
    ],
    out_shape=[
        jax.ShapeDtypeStruct((NPAD, 1), jnp.float32),
        jax.ShapeDtypeStruct((NPAD, 1), jnp.float32),
        jax.ShapeDtypeStruct((NPAD, D), jnp.float32),
    ],
)


def _update_body(p_ref, ni_ref, no_ref, w_ref, b_ref, g_ref):
    p = p_ref[0] + p_ref[1]
    a = p * ni_ref[...]
    h = jnp.dot(a, w_ref[...], preferred_element_type=jnp.float32) + b_ref[...]
    g_ref[...] = jnp.maximum(h, 0.0) * no_ref[...]


_update = pl.pallas_call(
    _update_body,
    grid=(GRIDP,),
    in_specs=[
        pl.BlockSpec((NC, BNP, H), lambda i: (0, i, 0)),
        pl.BlockSpec((BNP, 1), lambda i: (i, 0)),
        pl.BlockSpec((BNP, 1), lambda i: (i, 0)),
        pl.BlockSpec((H, H), lambda i: (0, 0)),
        pl.BlockSpec((1, H), lambda i: (0, 0)),
    ],
    out_specs=pl.BlockSpec((BNP, H), lambda i: (i, 0)),
    out_shape=jax.ShapeDtypeStruct((NPAD, H), jnp.float32),
)


def _final_body(p_ref, ni_ref, w4_ref, b4_ref, wv_ref, bv_ref, wc_ref, bc_ref,
                out_ref, num_ref, den_ref):
    i = pl.program_id(0)
    p = p_ref[0] + p_ref[1]
    a = p * ni_ref[...]
    h = jnp.dot(a, w4_ref[...], preferred_element_type=jnp.float32) + b4_ref[...]
    h = jnp.maximum(h, 0.0)
    logits = jnp.dot(h, wv_ref[...], preferred_element_type=jnp.float32) + bv_ref[...]
    w = jax.nn.sigmoid(logits)
    s_num = jnp.sum(w * h, axis=0, keepdims=True)
    s_den = jnp.sum(w)

    @pl.when(i == 0)
    def _():
        num_ref[...] = s_num
        den_ref[0] = s_den

    @pl.when(i > 0)
    def _():
        num_ref[...] = num_ref[...] + s_num
        den_ref[0] = den_ref[0] + s_den

    @pl.when(i == GRID - 1)
    def _():
        hg = num_ref[...] / jnp.maximum(den_ref[0], 1e-9)
        out_ref[...] = jnp.dot(hg, wc_ref[...], preferred_element_type=jnp.float32) + bc_ref[...]


_final = pl.pallas_call(
    _final_body,
    grid=(GRID,),
    in_specs=[
        pl.BlockSpec((NC, BN, H), lambda i: (0, i, 0)),
        pl.BlockSpec((BN, 1), lambda i: (i, 0)),
        pl.BlockSpec((H, H), lambda i: (0, 0)),
        pl.BlockSpec((1, H), lambda i: (0, 0)),
        pl.BlockSpec((H, 1), lambda i: (0, 0)),
        pl.BlockSpec((1, 1), lambda i: (0, 0)),
        pl.BlockSpec((H, C), lambda i: (0, 0)),
        pl.BlockSpec((1, C), lambda i: (0, 0)),
    ],
    out_specs=pl.BlockSpec((1, C), lambda i: (0, 0)),
    out_shape=jax.ShapeDtypeStruct((1, C), jnp.float32),
    scratch_shapes=[
        pltpu.VMEM((1, H), jnp.float32),
        pltpu.SMEM((1,), jnp.float32),
    ],
)


def kernel(x, edge_index, W1, b1, W2, b2, W3, b3, W4, b4, Wv, bv, Wc, bc):
    pad = jnp.full((EPAD - E,), PADROW, dtype=jnp.int32)
    src = jnp.concatenate([edge_index[0], pad])
    dst = jnp.concatenate([edge_index[1], pad])
    xp = jnp.concatenate([x, jnp.zeros((NPAD - N, D), jnp.float32)], axis=0)
    rowids = jnp.arange(NPAD, dtype=jnp.int32)
    onesrow = jnp.ones((CHUNK, H), jnp.float32)
    zrow = jnp.zeros((CHUNK, H), jnp.float32)

    dego = _deg_kernel(src, onesrow, zrow, rowids)
    degi = _deg_kernel(dst, onesrow, zrow, rowids)
    no, ni, g = _prep(dego, degi, xp)
    p = _agg_kernel(g, src, dst, zrow, rowids)
    g = _update(p, ni, no, W1, b1.reshape(1, H))
    p = _agg_kernel(g, src, dst, zrow, rowids)
    g = _update(p, ni, no, W2, b2.reshape(1, H))
    p = _agg_kernel(g, src, dst, zrow, rowids)
    g = _update(p, ni, no, W3, b3.reshape(1, H))
    p = _agg_kernel(g, src, dst, zrow, rowids)
    out = _final(p, ni, W4, b4.reshape(1, H), Wv, bv.reshape(1, 1),
                 Wc, bc.reshape(1, C))
    return out


# speedup vs baseline: 1.7969x; 1.7969x over previous
import functools

import jax
import jax.numpy as jnp
from jax import lax
from jax.experimental import pallas as pl
from jax.experimental.pallas import tpu as pltpu
from jax.experimental.pallas import tpu_sc as plsc

N = 10000
E = 320000
D = 128
H = 128
C = 10

NC = 2
NS = 16
NW = NC * NS
CHUNK = 128
JMAX = 80
EPAD = JMAX * NW * CHUNK
NPAD = 10240
PADROW = 10016
TROWS = NPAD // NS
RCH = TROWS // CHUNK

BNP = 640
GRIDP = NPAD // BNP
BN = 1000
GRID = N // BN

_mesh = plsc.VectorSubcoreMesh(
    core_axis_name="c", subcore_axis_name="s", num_cores=NC, num_subcores=NS
)


@functools.partial(
    pl.kernel,
    out_type=jax.ShapeDtypeStruct((NC, NPAD, H), jnp.float32),
    mesh=_mesh,
    scratch_types=[
        pltpu.VMEM((CHUNK,), jnp.int32),
        pltpu.VMEM((CHUNK,), jnp.int32),
        pltpu.VMEM((CHUNK, H), jnp.float32),
        pltpu.VMEM_SHARED((NPAD, H), jnp.float32),
    ],
)
def _deg_kernel(idx_hbm, ones_hbm, zero_hbm, rowid_hbm, out_hbm,
                eidx0, zidx, ones_v, acc):
    cid = lax.axis_index("c")
    sid = lax.axis_index("s")
    wid = sid * NC + cid
    base = sid * TROWS
    pltpu.sync_copy(zero_hbm, ones_v.at[pl.ds(0, CHUNK)])
    for r in range(RCH):
        pltpu.sync_copy(rowid_hbm.at[pl.ds(base + r * CHUNK, CHUNK)], zidx)
        pltpu.sync_copy(ones_v.at[pl.ds(0, CHUNK)], acc.at[zidx])
    pltpu.sync_copy(ones_hbm, ones_v)
    plsc.subcore_barrier()

    def body(j, carry):
        off = (j * NW + wid) * CHUNK
        pltpu.sync_copy(idx_hbm.at[pl.ds(off, CHUNK)], eidx0)
        pltpu.sync_copy(ones_v, acc.at[eidx0], add=True)
        return carry

    lax.fori_loop(0, JMAX, body, 0)
    plsc.subcore_barrier()
    for r in range(RCH):
        pltpu.sync_copy(rowid_hbm.at[pl.ds(base + r * CHUNK, CHUNK)], zidx)
        pltpu.sync_copy(acc.at[zidx], ones_v.at[pl.ds(0, CHUNK)])
        pltpu.sync_copy(ones_v.at[pl.ds(0, CHUNK)], out_hbm.at[cid, pl.ds(base + r * CHUNK, CHUNK)])


@functools.partial(
    pl.kernel,
    out_type=jax.ShapeDtypeStruct((NC, NPAD, H), jnp.float32),
    mesh=_mesh,
    scratch_types=[
        pltpu.VMEM((CHUNK,), jnp.int32),
        pltpu.VMEM((CHUNK,), jnp.int32),
        pltpu.VMEM((CHUNK,), jnp.int32),
        pltpu.VMEM((CHUNK, H), jnp.float32),
        pltpu.VMEM_SHARED((NPAD, H), jnp.float32),
        pltpu.SemaphoreType.DMA,
    ],
)
def _agg_kernel(g_hbm, src_hbm, dst_hbm, zero_hbm, rowid_hbm, out_hbm,
                sidx, didx, zidx, rows_v, acc, gsem):
    cid = lax.axis_index("c")
    sid = lax.axis_index("s")
    wid = sid * NC + cid
    base = sid * TROWS
    pltpu.sync_copy(zero_hbm, rows_v)
    for r in range(RCH):
        pltpu.sync_copy(rowid_hbm.at[pl.ds(base + r * CHUNK, CHUNK)], zidx)
        pltpu.sync_copy(rows_v, acc.at[zidx])
    plsc.subcore_barrier()

    def body(j, carry):
        off = (j * NW + wid) * CHUNK
        pltpu.sync_copy(src_hbm.at[pl.ds(off, CHUNK)], sidx)
        pltpu.sync_copy(dst_hbm.at[pl.ds(off, CHUNK)], didx)
        pltpu.async_copy(g_hbm.at[sidx], rows_v, gsem).wait()
        pltpu.sync_copy(rows_v, acc.at[didx], add=True)
        return carry

    lax.fori_loop(0, JMAX, body, 0)
    plsc.subcore_barrier()
    for r in range(RCH):
        pltpu.sync_copy(rowid_hbm.at[pl.ds(base + r * CHUNK, CHUNK)], zidx)
        pltpu.sync_copy(acc.at[zidx], rows_v.at[pl.ds(0, CHUNK)])
        pltpu.sync_copy(rows_v.at[pl.ds(0, CHUNK)], out_hbm.at[cid, pl.ds(base + r * CHUNK, CHUNK)])


def _prep_body(dego_ref, degi_ref, x_ref, no_ref, ni_ref, g_ref):
    do = dego_ref[0, :, :1] + dego_ref[1, :, :1]
    di = degi_ref[0, :, :1] + degi_ref[1, :, :1]
    no = jnp.where(do > 0, lax.rsqrt(jnp.maximum(do, 1.0)), 0.0)
    ni = jnp.where(di > 0, lax.rsqrt(jnp.maximum(di, 1.0)), 0.0)
    no_ref[...] = no
    ni_ref[...] = ni
    g_ref[...] = x_ref[...] * no


_prep = pl.pallas_call(
    _prep_body,
    grid=(GRIDP,),
    in_specs=[
        pl.BlockSpec((NC, BNP, H), lambda i: (0, i, 0)),
        pl.BlockSpec((NC, BNP, H), lambda i: (0, i, 0)),
        pl.BlockSpec((BNP, D), lambda i: (i, 0)),
    ],
    out_specs=[
        pl.BlockSpec((BNP, 1), lambda i: (i, 0)),
        pl.BlockSpec((BNP, 1), lambda i: (i, 0)),
        pl.BlockSpec((BNP, D), lambda i: (i, 0)),
    ],
    out_shape=[
        jax.ShapeDtypeStruct((NPAD, 1), jnp.float32),
        jax.ShapeDtypeStruct((NPAD, 1), jnp.float32),
        jax.ShapeDtypeStruct((NPAD, D), jnp.float32),
    ],
)


def _update_body(p_ref, ni_ref, no_ref, w_ref, b_ref, g_ref):
    p = p_ref[0] + p_ref[1]
    a = p * ni_ref[...]
    h = jnp.dot(a, w_ref[...], preferred_element_type=jnp.float32) + b_ref[...]
    g_ref[...] = jnp.maximum(h, 0.0) * no_ref[...]


_update = pl.pallas_call(
    _update_body,
    grid=(GRIDP,),
    in_specs=[
        pl.BlockSpec((NC, BNP, H), lambda i: (0, i, 0)),
        pl.BlockSpec((BNP, 1), lambda i: (i, 0)),
        pl.BlockSpec((BNP, 1), lambda i: (i, 0)),
        pl.BlockSpec((H, H), lambda i: (0, 0)),
        pl.BlockSpec((1, H), lambda i: (0, 0)),
    ],
    out_specs=pl.BlockSpec((BNP, H), lambda i: (i, 0)),
    out_shape=jax.ShapeDtypeStruct((NPAD, H), jnp.float32),
)


def _final_body(p_ref, ni_ref, w4_ref, b4_ref, wv_ref, bv_ref, wc_ref, bc_ref,
                out_ref, num_ref, den_ref):
    i = pl.program_id(0)
    p = p_ref[0] + p_ref[1]
    a = p * ni_ref[...]
    h = jnp.dot(a, w4_ref[...], preferred_element_type=jnp.float32) + b4_ref[...]
    h = jnp.maximum(h, 0.0)
    logits = jnp.dot(h, wv_ref[...], preferred_element_type=jnp.float32) + bv_ref[...]
    w = jax.nn.sigmoid(logits)
    s_num = jnp.sum(w * h, axis=0, keepdims=True)
    s_den = jnp.sum(w)

    @pl.when(i == 0)
    def _():
        num_ref[...] = s_num
        den_ref[0] = s_den

    @pl.when(i > 0)
    def _():
        num_ref[...] = num_ref[...] + s_num
        den_ref[0] = den_ref[0] + s_den

    @pl.when(i == GRID - 1)
    def _():
        hg = num_ref[...] / jnp.maximum(den_ref[0], 1e-9)
        out_ref[...] = jnp.dot(hg, wc_ref[...], preferred_element_type=jnp.float32) + bc_ref[...]


_final = pl.pallas_call(
    _final_body,
    grid=(GRID,),
    in_specs=[
        pl.BlockSpec((NC, BN, H), lambda i: (0, i, 0)),
        pl.BlockSpec((BN, 1), lambda i: (i, 0)),
        pl.BlockSpec((H, H), lambda i: (0, 0)),
        pl.BlockSpec((1, H), lambda i: (0, 0)),
        pl.BlockSpec((H, 1), lambda i: (0, 0)),
        pl.BlockSpec((1, 1), lambda i: (0, 0)),
        pl.BlockSpec((H, C), lambda i: (0, 0)),
        pl.BlockSpec((1, C), lambda i: (0, 0)),
    ],
    out_specs=pl.BlockSpec((1, C), lambda i: (0, 0)),
    out_shape=jax.ShapeDtypeStruct((1, C), jnp.float32),
    scratch_shapes=[
        pltpu.VMEM((1, H), jnp.float32),
        pltpu.SMEM((1,), jnp.float32),
    ],
)


def kernel(x, edge_index, W1, b1, W2, b2, W3, b3, W4, b4, Wv, bv, Wc, bc):
    pad = PADROW + (jnp.arange(EPAD - E, dtype=jnp.int32) % (NPAD - PADROW))
    src = jnp.concatenate([edge_index[0], pad])
    dst = jnp.concatenate([edge_index[1], pad])
    xp = jnp.concatenate([x, jnp.zeros((NPAD - N, D), jnp.float32)], axis=0)
    rowids = jnp.arange(NPAD, dtype=jnp.int32)
    onesrow = jnp.ones((CHUNK, H), jnp.float32)
    zrow = jnp.zeros((CHUNK, H), jnp.float32)

    dego = _deg_kernel(src, onesrow, zrow, rowids)
    degi = _deg_kernel(dst, onesrow, zrow, rowids)
    no, ni, g = _prep(dego, degi, xp)
    p = _agg_kernel(g, src, dst, zrow, rowids)
    g = _update(p, ni, no, W1, b1.reshape(1, H))
    p = _agg_kernel(g, src, dst, zrow, rowids)
    g = _update(p, ni, no, W2, b2.reshape(1, H))
    p = _agg_kernel(g, src, dst, zrow, rowids)
    g = _update(p, ni, no, W3, b3.reshape(1, H))
    p = _agg_kernel(g, src, dst, zrow, rowids)
    out = _final(p, ni, W4, b4.reshape(1, H), Wv, bv.reshape(1, 1),
                 Wc, bc.reshape(1, C))
    return out


# speedup vs baseline: 2.2830x; 1.2705x over previous
import functools

import jax
import jax.numpy as jnp
from jax import lax
from jax.experimental import pallas as pl
from jax.experimental.pallas import tpu as pltpu
from jax.experimental.pallas import tpu_sc as plsc

N = 10000
E = 320000
D = 128
H = 128
C = 10

NC = 2
NS = 16
NW = NC * NS
CHUNK = 128
JMAX = 80
EPAD = JMAX * NW * CHUNK
NPAD = 10240
PADROW = 10016
TROWS = NPAD // NS
RCH = TROWS // CHUNK

BNP = 640
GRIDP = NPAD // BNP
BN = 1000
GRID = N // BN

_mesh = plsc.VectorSubcoreMesh(
    core_axis_name="c", subcore_axis_name="s", num_cores=NC, num_subcores=NS
)


@functools.partial(
    pl.kernel,
    out_type=jax.ShapeDtypeStruct((NC, NPAD, H), jnp.float32),
    mesh=_mesh,
    scratch_types=[
        pltpu.VMEM((CHUNK,), jnp.int32),
        pltpu.VMEM((CHUNK,), jnp.int32),
        pltpu.VMEM((CHUNK, H), jnp.float32),
        pltpu.VMEM_SHARED((NPAD, H), jnp.float32),
    ],
)
def _deg_kernel(idx_hbm, ones_hbm, zero_hbm, rowid_hbm, out_hbm,
                eidx0, zidx, ones_v, acc):
    cid = lax.axis_index("c")
    sid = lax.axis_index("s")
    wid = sid * NC + cid
    base = sid * TROWS
    pltpu.sync_copy(zero_hbm, ones_v.at[pl.ds(0, CHUNK)])
    for r in range(RCH):
        pltpu.sync_copy(rowid_hbm.at[pl.ds(base + r * CHUNK, CHUNK)], zidx)
        pltpu.sync_copy(ones_v.at[pl.ds(0, CHUNK)], acc.at[zidx])
    pltpu.sync_copy(ones_hbm, ones_v)
    plsc.subcore_barrier()

    def body(j, carry):
        off = (j * NW + wid) * CHUNK
        pltpu.sync_copy(idx_hbm.at[pl.ds(off, CHUNK)], eidx0)
        pltpu.sync_copy(ones_v, acc.at[eidx0], add=True)
        return carry

    lax.fori_loop(0, JMAX, body, 0)
    plsc.subcore_barrier()
    for r in range(RCH):
        pltpu.sync_copy(rowid_hbm.at[pl.ds(base + r * CHUNK, CHUNK)], zidx)
        pltpu.sync_copy(acc.at[zidx], ones_v.at[pl.ds(0, CHUNK)])
        pltpu.sync_copy(ones_v.at[pl.ds(0, CHUNK)], out_hbm.at[cid, pl.ds(base + r * CHUNK, CHUNK)])


@functools.partial(
    pl.kernel,
    out_type=jax.ShapeDtypeStruct((NC, NPAD, H), jnp.float32),
    mesh=_mesh,
    scratch_types=[
        pltpu.VMEM((2, CHUNK), jnp.int32),
        pltpu.VMEM((2, CHUNK), jnp.int32),
        pltpu.VMEM((CHUNK,), jnp.int32),
        pltpu.VMEM((CHUNK, H), jnp.float32),
        pltpu.VMEM_SHARED((NPAD, H), jnp.float32),
        pltpu.SemaphoreType.DMA,
        pltpu.SemaphoreType.DMA,
        pltpu.SemaphoreType.DMA,
    ],
)
def _agg_kernel(g_hbm, edges_hbm, zero_hbm, rowid_hbm, out_hbm,
                eb0, eb1, zidx, rows_v, acc, gsem, isem0, isem1):
    cid = lax.axis_index("c")
    sid = lax.axis_index("s")
    wid = sid * NC + cid
    base = sid * TROWS
    pltpu.sync_copy(zero_hbm, rows_v)
    for r in range(RCH):
        pltpu.sync_copy(rowid_hbm.at[pl.ds(base + r * CHUNK, CHUNK)], zidx)
        pltpu.sync_copy(rows_v, acc.at[zidx])
    plsc.subcore_barrier()

    i0 = pltpu.async_copy(edges_hbm.at[wid], eb0, isem0)
    i1 = pltpu.async_copy(edges_hbm.at[NW + wid], eb1, isem1)

    def step(j, eb, idesc):
        idesc.wait()
        pltpu.async_copy(g_hbm.at[eb.at[0]], rows_v, gsem).wait()
        pltpu.sync_copy(rows_v, acc.at[eb.at[1]], add=True)

    def body(jj, carry):
        j0 = 2 * jj
        step(j0, eb0, i0)
        pltpu.async_copy(edges_hbm.at[(j0 + 2) * NW + wid], eb0, isem0)
        step(j0 + 1, eb1, i1)
        pltpu.async_copy(edges_hbm.at[(j0 + 3) * NW + wid], eb1, isem1)
        return carry

    lax.fori_loop(0, JMAX // 2 - 1, body, 0)
    step(JMAX - 2, eb0, i0)
    step(JMAX - 1, eb1, i1)
    plsc.subcore_barrier()
    for r in range(RCH):
        pltpu.sync_copy(rowid_hbm.at[pl.ds(base + r * CHUNK, CHUNK)], zidx)
        pltpu.sync_copy(acc.at[zidx], rows_v.at[pl.ds(0, CHUNK)])
        pltpu.sync_copy(rows_v.at[pl.ds(0, CHUNK)], out_hbm.at[cid, pl.ds(base + r * CHUNK, CHUNK)])


def _prep_body(dego_ref, degi_ref, x_ref, no_ref, ni_ref, g_ref):
    do = dego_ref[0, :, :1] + dego_ref[1, :, :1]
    di = degi_ref[0, :, :1] + degi_ref[1, :, :1]
    no = jnp.where(do > 0, lax.rsqrt(jnp.maximum(do, 1.0)), 0.0)
    ni = jnp.where(di > 0, lax.rsqrt(jnp.maximum(di, 1.0)), 0.0)
    no_ref[...] = no
    ni_ref[...] = ni
    g_ref[...] = x_ref[...] * no


_prep = pl.pallas_call(
    _prep_body,
    grid=(GRIDP,),
    in_specs=[
        pl.BlockSpec((NC, BNP, H), lambda i: (0, i, 0)),
        pl.BlockSpec((NC, BNP, H), lambda i: (0, i, 0)),
        pl.BlockSpec((BNP, D), lambda i: (i, 0)),
    ],
    out_specs=[
        pl.BlockSpec((BNP, 1), lambda i: (i, 0)),
        pl.BlockSpec((BNP, 1), lambda i: (i, 0)),
        pl.BlockSpec((BNP, D), lambda i: (i, 0)),
    ],
    out_shape=[
        jax.ShapeDtypeStruct((NPAD, 1), jnp.float32),
        jax.ShapeDtypeStruct((NPAD, 1), jnp.float32),
        jax.ShapeDtypeStruct((NPAD, D), jnp.float32),
    ],
)


def _update_body(p_ref, ni_ref, no_ref, w_ref, b_ref, g_ref):
    p = p_ref[0] + p_ref[1]
    a = p * ni_ref[...]
    h = jnp.dot(a, w_ref[...], preferred_element_type=jnp.float32) + b_ref[...]
    g_ref[...] = jnp.maximum(h, 0.0) * no_ref[...]


_update = pl.pallas_call(
    _update_body,
    grid=(GRIDP,),
    in_specs=[
        pl.BlockSpec((NC, BNP, H), lambda i: (0, i, 0)),
        pl.BlockSpec((BNP, 1), lambda i: (i, 0)),
        pl.BlockSpec((BNP, 1), lambda i: (i, 0)),
        pl.BlockSpec((H, H), lambda i: (0, 0)),
        pl.BlockSpec((1, H), lambda i: (0, 0)),
    ],
    out_specs=pl.BlockSpec((BNP, H), lambda i: (i, 0)),
    out_shape=jax.ShapeDtypeStruct((NPAD, H), jnp.float32),
)


def _final_body(p_ref, ni_ref, w4_ref, b4_ref, wv_ref, bv_ref, wc_ref, bc_ref,
                out_ref, num_ref, den_ref):
    i = pl.program_id(0)
    p = p_ref[0] + p_ref[1]
    a = p * ni_ref[...]
    h = jnp.dot(a, w4_ref[...], preferred_element_type=jnp.float32) + b4_ref[...]
    h = jnp.maximum(h, 0.0)
    logits = jnp.dot(h, wv_ref[...], preferred_element_type=jnp.float32) + bv_ref[...]
    w = jax.nn.sigmoid(logits)
    s_num = jnp.sum(w * h, axis=0, keepdims=True)
    s_den = jnp.sum(w)

    @pl.when(i == 0)
    def _():
        num_ref[...] = s_num
        den_ref[0] = s_den

    @pl.when(i > 0)
    def _():
        num_ref[...] = num_ref[...] + s_num
        den_ref[0] = den_ref[0] + s_den

    @pl.when(i == GRID - 1)
    def _():
        hg = num_ref[...] / jnp.maximum(den_ref[0], 1e-9)
        out_ref[...] = jnp.dot(hg, wc_ref[...], preferred_element_type=jnp.float32) + bc_ref[...]


_final = pl.pallas_call(
    _final_body,
    grid=(GRID,),
    in_specs=[
        pl.BlockSpec((NC, BN, H), lambda i: (0, i, 0)),
        pl.BlockSpec((BN, 1), lambda i: (i, 0)),
        pl.BlockSpec((H, H), lambda i: (0, 0)),
        pl.BlockSpec((1, H), lambda i: (0, 0)),
        pl.BlockSpec((H, 1), lambda i: (0, 0)),
        pl.BlockSpec((1, 1), lambda i: (0, 0)),
        pl.BlockSpec((H, C), lambda i: (0, 0)),
        pl.BlockSpec((1, C), lambda i: (0, 0)),
    ],
    out_specs=pl.BlockSpec((1, C), lambda i: (0, 0)),
    out_shape=jax.ShapeDtypeStruct((1, C), jnp.float32),
    scratch_shapes=[
        pltpu.VMEM((1, H), jnp.float32),
        pltpu.SMEM((1,), jnp.float32),
    ],
)


def kernel(x, edge_index, W1, b1, W2, b2, W3, b3, W4, b4, Wv, bv, Wc, bc):
    pad = PADROW + (jnp.arange(EPAD - E, dtype=jnp.int32) % (NPAD - PADROW))
    src = jnp.concatenate([edge_index[0], pad])
    dst = jnp.concatenate([edge_index[1], pad])
    xp = jnp.concatenate([x, jnp.zeros((NPAD - N, D), jnp.float32)], axis=0)
    rowids = jnp.arange(NPAD, dtype=jnp.int32)
    onesrow = jnp.ones((CHUNK, H), jnp.float32)
    zrow = jnp.zeros((CHUNK, H), jnp.float32)

    edges3 = jnp.stack([src.reshape(-1, CHUNK), dst.reshape(-1, CHUNK)], axis=1)

    dego = _deg_kernel(src, onesrow, zrow, rowids)
    degi = _deg_kernel(dst, onesrow, zrow, rowids)
    no, ni, g = _prep(dego, degi, xp)
    p = _agg_kernel(g, edges3, zrow, rowids)
    g = _update(p, ni, no, W1, b1.reshape(1, H))
    p = _agg_kernel(g, edges3, zrow, rowids)
    g = _update(p, ni, no, W2, b2.reshape(1, H))
    p = _agg_kernel(g, edges3, zrow, rowids)
    g = _update(p, ni, no, W3, b3.reshape(1, H))
    p = _agg_kernel(g, edges3, zrow, rowids)
    out = _final(p, ni, W4, b4.reshape(1, H), Wv, bv.reshape(1, 1),
                 Wc, bc.reshape(1, C))
    return out


# speedup vs baseline: 2.4359x; 1.0670x over previous
import functools

import jax
import jax.numpy as jnp
from jax import lax
from jax.experimental import pallas as pl
from jax.experimental.pallas import tpu as pltpu
from jax.experimental.pallas import tpu_sc as plsc

N = 10000
E = 320000
D = 128
H = 128
C = 10

NC = 2
NS = 16
NW = NC * NS
CHUNK = 128
JMAX = 80
EPAD = JMAX * NW * CHUNK
NPAD = 10240
PADROW = 10016
TROWS = NPAD // NS
RCH = TROWS // CHUNK

BNP = 640
GRIDP = NPAD // BNP
BN = 1000
GRID = N // BN

_mesh = plsc.VectorSubcoreMesh(
    core_axis_name="c", subcore_axis_name="s", num_cores=NC, num_subcores=NS
)


@functools.partial(
    pl.kernel,
    out_type=jax.ShapeDtypeStruct((NC, NPAD, H), jnp.float32),
    mesh=_mesh,
    scratch_types=[
        pltpu.VMEM((CHUNK,), jnp.int32),
        pltpu.VMEM((CHUNK,), jnp.int32),
        pltpu.VMEM((CHUNK,), jnp.int32),
        pltpu.VMEM((CHUNK, H), jnp.float32),
        pltpu.VMEM_SHARED((NPAD, H), jnp.float32),
        pltpu.SemaphoreType.DMA,
        pltpu.SemaphoreType.DMA,
    ],
)
def _deg_kernel(idx_hbm, ones_hbm, zero_hbm, rowid_hbm, out_hbm,
                eidx0, eidx1, zidx, ones_v, acc, isem0, isem1):
    cid = lax.axis_index("c")
    sid = lax.axis_index("s")
    wid = sid * NC + cid
    base = sid * TROWS
    pltpu.sync_copy(zero_hbm, ones_v.at[pl.ds(0, CHUNK)])
    for r in range(RCH):
        pltpu.sync_copy(rowid_hbm.at[pl.ds(base + r * CHUNK, CHUNK)], zidx)
        pltpu.sync_copy(ones_v.at[pl.ds(0, CHUNK)], acc.at[zidx])
    pltpu.sync_copy(ones_hbm, ones_v)
    plsc.subcore_barrier()

    i0 = pltpu.async_copy(idx_hbm.at[pl.ds(wid * CHUNK, CHUNK)], eidx0, isem0)
    i1 = pltpu.async_copy(idx_hbm.at[pl.ds((NW + wid) * CHUNK, CHUNK)], eidx1, isem1)

    def body(jj, carry):
        j0 = 2 * jj
        i0.wait()
        pltpu.sync_copy(ones_v, acc.at[eidx0], add=True)
        pltpu.async_copy(idx_hbm.at[pl.ds(((j0 + 2) * NW + wid) * CHUNK, CHUNK)], eidx0, isem0)
        i1.wait()
        pltpu.sync_copy(ones_v, acc.at[eidx1], add=True)
        pltpu.async_copy(idx_hbm.at[pl.ds(((j0 + 3) * NW + wid) * CHUNK, CHUNK)], eidx1, isem1)
        return carry

    lax.fori_loop(0, JMAX // 2 - 1, body, 0)
    i0.wait()
    pltpu.sync_copy(ones_v, acc.at[eidx0], add=True)
    i1.wait()
    pltpu.sync_copy(ones_v, acc.at[eidx1], add=True)
    plsc.subcore_barrier()
    for r in range(RCH):
        pltpu.sync_copy(rowid_hbm.at[pl.ds(base + r * CHUNK, CHUNK)], zidx)
        pltpu.sync_copy(acc.at[zidx], ones_v.at[pl.ds(0, CHUNK)])
        pltpu.sync_copy(ones_v.at[pl.ds(0, CHUNK)], out_hbm.at[cid, pl.ds(base + r * CHUNK, CHUNK)])


@functools.partial(
    pl.kernel,
    out_type=jax.ShapeDtypeStruct((NC, NPAD, H), jnp.float32),
    mesh=_mesh,
    scratch_types=[
        pltpu.VMEM((2, CHUNK), jnp.int32),
        pltpu.VMEM((2, CHUNK), jnp.int32),
        pltpu.VMEM((CHUNK,), jnp.int32),
        pltpu.VMEM((CHUNK, H), jnp.float32),
        pltpu.VMEM_SHARED((NPAD, H), jnp.float32),
        pltpu.SemaphoreType.DMA,
        pltpu.SemaphoreType.DMA,
        pltpu.SemaphoreType.DMA,
    ],
)
def _agg_kernel(g_hbm, edges_hbm, zero_hbm, rowid_hbm, out_hbm,
                eb0, eb1, zidx, rows_v, acc, gsem, isem0, isem1):
    cid = lax.axis_index("c")
    sid = lax.axis_index("s")
    wid = sid * NC + cid
    base = sid * TROWS
    pltpu.sync_copy(zero_hbm, rows_v)
    for r in range(RCH):
        pltpu.sync_copy(rowid_hbm.at[pl.ds(base + r * CHUNK, CHUNK)], zidx)
        pltpu.sync_copy(rows_v, acc.at[zidx])
    plsc.subcore_barrier()

    i0 = pltpu.async_copy(edges_hbm.at[wid], eb0, isem0)
    i1 = pltpu.async_copy(edges_hbm.at[NW + wid], eb1, isem1)

    def step(j, eb, idesc):
        idesc.wait()
        pltpu.async_copy(g_hbm.at[eb.at[0]], rows_v, gsem).wait()
        pltpu.sync_copy(rows_v, acc.at[eb.at[1]], add=True)

    def body(jj, carry):
        j0 = 2 * jj
        step(j0, eb0, i0)
        pltpu.async_copy(edges_hbm.at[(j0 + 2) * NW + wid], eb0, isem0)
        step(j0 + 1, eb1, i1)
        pltpu.async_copy(edges_hbm.at[(j0 + 3) * NW + wid], eb1, isem1)
        return carry

    lax.fori_loop(0, JMAX // 2 - 1, body, 0)
    step(JMAX - 2, eb0, i0)
    step(JMAX - 1, eb1, i1)
    plsc.subcore_barrier()
    for r in range(RCH):
        pltpu.sync_copy(rowid_hbm.at[pl.ds(base + r * CHUNK, CHUNK)], zidx)
        pltpu.sync_copy(acc.at[zidx], rows_v.at[pl.ds(0, CHUNK)])
        pltpu.sync_copy(rows_v.at[pl.ds(0, CHUNK)], out_hbm.at[cid, pl.ds(base + r * CHUNK, CHUNK)])


def _prep_body(dego_ref, degi_ref, x_ref, no_ref, ni_ref, g_ref):
    do = dego_ref[0, :, :1] + dego_ref[1, :, :1]
    di = degi_ref[0, :, :1] + degi_ref[1, :, :1]
    no = jnp.where(do > 0, lax.rsqrt(jnp.maximum(do, 1.0)), 0.0)
    ni = jnp.where(di > 0, lax.rsqrt(jnp.maximum(di, 1.0)), 0.0)
    no_ref[...] = no
    ni_ref[...] = ni
    g_ref[...] = x_ref[...] * no


_prep = pl.pallas_call(
    _prep_body,
    grid=(GRIDP,),
    in_specs=[
        pl.BlockSpec((NC, BNP, H), lambda i: (0, i, 0)),
        pl.BlockSpec((NC, BNP, H), lambda i: (0, i, 0)),
        pl.BlockSpec((BNP, D), lambda i: (i, 0)),
    ],
    out_specs=[
        pl.BlockSpec((BNP, 1), lambda i: (i, 0)),
        pl.BlockSpec((BNP, 1), lambda i: (i, 0)),
        pl.BlockSpec((BNP, D), lambda i: (i, 0)),
    ],
    out_shape=[
        jax.ShapeDtypeStruct((NPAD, 1), jnp.float32),
        jax.ShapeDtypeStruct((NPAD, 1), jnp.float32),
        jax.ShapeDtypeStruct((NPAD, D), jnp.float32),
    ],
)


def _update_body(p_ref, ni_ref, no_ref, w_ref, b_ref, g_ref):
    p = p_ref[0] + p_ref[1]
    a = p * ni_ref[...]
    h = jnp.dot(a, w_ref[...], preferred_element_type=jnp.float32) + b_ref[...]
    g_ref[...] = jnp.maximum(h, 0.0) * no_ref[...]


_update = pl.pallas_call(
    _update_body,
    grid=(GRIDP,),
    in_specs=[
        pl.BlockSpec((NC, BNP, H), lambda i: (0, i, 0)),
        pl.BlockSpec((BNP, 1), lambda i: (i, 0)),
        pl.BlockSpec((BNP, 1), lambda i: (i, 0)),
        pl.BlockSpec((H, H), lambda i: (0, 0)),
        pl.BlockSpec((1, H), lambda i: (0, 0)),
    ],
    out_specs=pl.BlockSpec((BNP, H), lambda i: (i, 0)),
    out_shape=jax.ShapeDtypeStruct((NPAD, H), jnp.float32),
)


def _final_body(p_ref, ni_ref, w4_ref, b4_ref, wv_ref, bv_ref, wc_ref, bc_ref,
                out_ref, num_ref, den_ref):
    i = pl.program_id(0)
    p = p_ref[0] + p_ref[1]
    a = p * ni_ref[...]
    h = jnp.dot(a, w4_ref[...], preferred_element_type=jnp.float32) + b4_ref[...]
    h = jnp.maximum(h, 0.0)
    logits = jnp.dot(h, wv_ref[...], preferred_element_type=jnp.float32) + bv_ref[...]
    w = jax.nn.sigmoid(logits)
    s_num = jnp.sum(w * h, axis=0, keepdims=True)
    s_den = jnp.sum(w)

    @pl.when(i == 0)
    def _():
        num_ref[...] = s_num
        den_ref[0] = s_den

    @pl.when(i > 0)
    def _():
        num_ref[...] = num_ref[...] + s_num
        den_ref[0] = den_ref[0] + s_den

    @pl.when(i == GRID - 1)
    def _():
        hg = num_ref[...] / jnp.maximum(den_ref[0], 1e-9)
        out_ref[...] = jnp.dot(hg, wc_ref[...], preferred_element_type=jnp.float32) + bc_ref[...]


_final = pl.pallas_call(
    _final_body,
    grid=(GRID,),
    in_specs=[
        pl.BlockSpec((NC, BN, H), lambda i: (0, i, 0)),
        pl.BlockSpec((BN, 1), lambda i: (i, 0)),
        pl.BlockSpec((H, H), lambda i: (0, 0)),
        pl.BlockSpec((1, H), lambda i: (0, 0)),
        pl.BlockSpec((H, 1), lambda i: (0, 0)),
        pl.BlockSpec((1, 1), lambda i: (0, 0)),
        pl.BlockSpec((H, C), lambda i: (0, 0)),
        pl.BlockSpec((1, C), lambda i: (0, 0)),
    ],
    out_specs=pl.BlockSpec((1, C), lambda i: (0, 0)),
    out_shape=jax.ShapeDtypeStruct((1, C), jnp.float32),
    scratch_shapes=[
        pltpu.VMEM((1, H), jnp.float32),
        pltpu.SMEM((1,), jnp.float32),
    ],
)


def kernel(x, edge_index, W1, b1, W2, b2, W3, b3, W4, b4, Wv, bv, Wc, bc):
    pad = PADROW + (jnp.arange(EPAD - E, dtype=jnp.int32) % (NPAD - PADROW))
    src = jnp.concatenate([edge_index[0], pad])
    dst = jnp.concatenate([edge_index[1], pad])
    xp = jnp.concatenate([x, jnp.zeros((NPAD - N, D), jnp.float32)], axis=0)
    rowids = jnp.arange(NPAD, dtype=jnp.int32)
    onesrow = jnp.ones((CHUNK, H), jnp.float32)
    zrow = jnp.zeros((CHUNK, H), jnp.float32)

    edges3 = jnp.stack([src.reshape(-1, CHUNK), dst.reshape(-1, CHUNK)], axis=1)

    dego = _deg_kernel(src, onesrow, zrow, rowids)
    degi = _deg_kernel(dst, onesrow, zrow, rowids)
    no, ni, g = _prep(dego, degi, xp)
    p = _agg_kernel(g, edges3, zrow, rowids)
    g = _update(p, ni, no, W1, b1.reshape(1, H))
    p = _agg_kernel(g, edges3, zrow, rowids)
    g = _update(p, ni, no, W2, b2.reshape(1, H))
    p = _agg_kernel(g, edges3, zrow, rowids)
    g = _update(p, ni, no, W3, b3.reshape(1, H))
    p = _agg_kernel(g, edges3, zrow, rowids)
    out = _final(p, ni, W4, b4.reshape(1, H), Wv, bv.reshape(1, 1),
                 Wc, bc.reshape(1, C))
    return out
